# Initial kernel scaffold; baseline (speedup 1.0000x reference)
import jax
import jax.numpy as jnp
from jax.experimental import pallas as pl


def kernel(indices, table):
    raise NotImplementedError("write your pallas kernel here")


# speedup vs baseline: 5.0226x; 5.0226x over previous
import functools

import jax
import jax.numpy as jnp
from jax import lax
from jax.experimental import pallas as pl
from jax.experimental.pallas import tpu as pltpu
from jax.experimental.pallas import tpu_sc as plsc

BATCH = 4096
SEQ_LEN = 200
VOCAB = 6
DIM = 64
TOTAL = BATCH * SEQ_LEN
PAIRS = TOTAL // 2

_info = plsc.get_sparse_core_info()
_NC, _NS = _info.num_cores, _info.num_subcores
_NW = _NC * _NS
_PER_W = PAIRS // _NW
_GCHUNK = 128
_NG = _PER_W // _GCHUNK


def _make_sc_gather():
    mesh = plsc.VectorSubcoreMesh(core_axis_name="c", subcore_axis_name="s")

    @functools.partial(
        pl.kernel,
        mesh=mesh,
        compiler_params=pltpu.CompilerParams(needs_layout_passes=False),
        out_type=jax.ShapeDtypeStruct((PAIRS, 2 * DIM), jnp.float32),
        scratch_types=[
            pltpu.VMEM((VOCAB, DIM), jnp.float32),
            pltpu.VMEM((VOCAB * VOCAB, 2 * DIM), jnp.float32),
            pltpu.VMEM_SHARED((VOCAB * VOCAB, 2 * DIM), jnp.float32),
            pltpu.VMEM((2 * _PER_W,), jnp.int32),
            pltpu.VMEM((_NG, _GCHUNK), jnp.int32),
            pltpu.VMEM((_GCHUNK, 2 * DIM), jnp.float32),
            pltpu.SemaphoreType.DMA,
        ],
    )
    def gather_kernel(table_hbm, idx_hbm, out_hbm,
                      tab_v, ptab_v, ptab_sh, idx_v, pidx_v, rows_v, sem):
        wid = lax.axis_index("s") * _NC + lax.axis_index("c")
        wbase = wid * _PER_W

        pltpu.sync_copy(table_hbm, tab_v)
        pltpu.sync_copy(idx_hbm.at[pl.ds(wbase * 2, 2 * _PER_W)], idx_v)

        nvec = DIM // 16
        tvregs = [[tab_v[a, pl.ds(k * 16, 16)] for k in range(nvec)]
                  for a in range(VOCAB)]
        for a in range(VOCAB):
            for b in range(VOCAB):
                p = a * VOCAB + b
                for k in range(nvec):
                    ptab_v[p, pl.ds(k * 16, 16)] = tvregs[a][k]
                    ptab_v[p, pl.ds(DIM + k * 16, 16)] = tvregs[b][k]

        @pl.when(lax.axis_index("s") == 0)
        def _():
            pltpu.sync_copy(ptab_v, ptab_sh)

        lanes2 = lax.iota(jnp.int32, 16) * 2

        def encode_row(r, _):
            for i in range(_GCHUNK // 16):
                base = r * (2 * _GCHUNK) + i * 32
                av = plsc.load_gather(idx_v, [lanes2 + base])
                bv = plsc.load_gather(idx_v, [lanes2 + (base + 1)])
                pidx_v[r, pl.ds(i * 16, 16)] = av * VOCAB + bv
            return 0

        lax.fori_loop(0, _NG, encode_row, 0)
        plsc.subcore_barrier()

        def gather_step(g, _):
            pltpu.async_copy(ptab_sh.at[pidx_v.at[g]], rows_v, sem).wait()
            pltpu.sync_copy(rows_v, out_hbm.at[pl.ds(wbase + g * _GCHUNK,
                                                     _GCHUNK)])
            return 0

        lax.fori_loop(0, _NG, gather_step, 0)

    return gather_kernel


_sc_gather = _make_sc_gather()


def kernel(indices, table):
    flat_idx = indices.reshape(-1).astype(jnp.int32)
    paired = _sc_gather(table, flat_idx)
    return paired.reshape(TOTAL, DIM)


# speedup vs baseline: 5.4997x; 1.0950x over previous
import functools

import jax
import jax.numpy as jnp
from jax import lax
from jax.experimental import pallas as pl
from jax.experimental.pallas import tpu as pltpu
from jax.experimental.pallas import tpu_sc as plsc

BATCH = 4096
SEQ_LEN = 200
VOCAB = 6
DIM = 64
TOTAL = BATCH * SEQ_LEN
PAIRS = TOTAL // 2

_info = plsc.get_sparse_core_info()
_NC, _NS = _info.num_cores, _info.num_subcores
_NW = _NC * _NS
_PER_W = PAIRS // _NW
_GCHUNK = 128
_NG = _PER_W // _GCHUNK
_NBUF = 4


def _make_sc_gather():
    mesh = plsc.VectorSubcoreMesh(core_axis_name="c", subcore_axis_name="s")

    @functools.partial(
        pl.kernel,
        mesh=mesh,
        compiler_params=pltpu.CompilerParams(needs_layout_passes=False),
        out_type=jax.ShapeDtypeStruct((PAIRS, 2 * DIM), jnp.float32),
        scratch_types=[
            pltpu.VMEM((VOCAB, DIM), jnp.float32),
            pltpu.VMEM((VOCAB * VOCAB, 2 * DIM), jnp.float32),
            pltpu.VMEM_SHARED((VOCAB * VOCAB, 2 * DIM), jnp.float32),
            pltpu.VMEM((2 * _PER_W,), jnp.int32),
            pltpu.VMEM((_NG, _GCHUNK), jnp.int32),
            pltpu.VMEM((_NBUF * _GCHUNK, 2 * DIM), jnp.float32),
        ] + [pltpu.SemaphoreType.DMA] * (2 * _NBUF),
    )
    def gather_kernel(table_hbm, idx_hbm, out_hbm,
                      tab_v, ptab_v, ptab_sh, idx_v, pidx_v, rows_v, *sems):
        gsems = sems[:_NBUF]
        ssems = sems[_NBUF:2 * _NBUF]
        wid = lax.axis_index("s") * _NC + lax.axis_index("c")
        wbase = wid * _PER_W

        pltpu.sync_copy(table_hbm, tab_v)
        pltpu.sync_copy(idx_hbm.at[pl.ds(wbase * 2, 2 * _PER_W)], idx_v)

        nvec = DIM // 16
        tvregs = [[tab_v[a, pl.ds(k * 16, 16)] for k in range(nvec)]
                  for a in range(VOCAB)]
        for a in range(VOCAB):
            for b in range(VOCAB):
                p = a * VOCAB + b
                for k in range(nvec):
                    ptab_v[p, pl.ds(k * 16, 16)] = tvregs[a][k]
                    ptab_v[p, pl.ds(DIM + k * 16, 16)] = tvregs[b][k]

        @pl.when(lax.axis_index("s") == 0)
        def _():
            pltpu.sync_copy(ptab_v, ptab_sh)

        lanes2 = lax.iota(jnp.int32, 16) * 2

        def encode_row(r, _):
            for i in range(_GCHUNK // 16):
                base = r * (2 * _GCHUNK) + i * 32
                av = plsc.load_gather(idx_v, [lanes2 + base])
                bv = plsc.load_gather(idx_v, [lanes2 + (base + 1)])
                pidx_v[r, pl.ds(i * 16, 16)] = av * VOCAB + bv
            return 0

        lax.fori_loop(0, _NG, encode_row, 0)
        plsc.subcore_barrier()

        bufs = [rows_v.at[pl.ds(b * _GCHUNK, _GCHUNK)] for b in range(_NBUF)]

        def start_gather(g):
            return pltpu.async_copy(ptab_sh.at[pidx_v.at[g]],
                                    bufs[g % _NBUF], gsems[g % _NBUF])

        def start_scatter(g):
            return pltpu.async_copy(
                bufs[g % _NBUF],
                out_hbm.at[pl.ds(wbase + g * _GCHUNK, _GCHUNK)],
                ssems[g % _NBUF])

        gcp, scp = {}, {}
        for g in range(_NG):
            if g >= _NBUF:
                scp[g - _NBUF].wait()
            gcp[g] = start_gather(g)
            gp = g - (_NBUF - 1)
            if gp >= 0:
                gcp[gp].wait()
                scp[gp] = start_scatter(gp)
        for gp in range(_NG - (_NBUF - 1), _NG):
            gcp[gp].wait()
            scp[gp] = start_scatter(gp)
        for g in range(max(0, _NG - _NBUF), _NG):
            scp[g].wait()

    return gather_kernel


_sc_gather = _make_sc_gather()


def kernel(indices, table):
    flat_idx = indices.reshape(-1).astype(jnp.int32)
    paired = _sc_gather(table, flat_idx)
    return paired.reshape(TOTAL, DIM)
